# baseline (speedup 1.0000x reference)
import dataclasses
import functools

import jax
import jax.numpy as jnp
from jax import lax
from jax.experimental import pallas as pl
from jax.experimental.pallas import tpu as pltpu
from jax.experimental.pallas import tpu_sc as plsc

N = 50000
E = 1600000
NC = 2
NS = 16
NW = NC * NS
EPW = E // NW
CHA = 2000
CHB = 2000
NPAD = 50176
NP2 = 2 * NPAD
BT = 6400
BTC = 6272


def _sc_compiler_params():
    cp = pltpu.CompilerParams()
    if "needs_layout_passes" in pltpu.CompilerParams.__dataclass_fields__:
        cp = dataclasses.replace(cp, needs_layout_passes=False)
    return cp


def _mass_product(node_flat, src, dst):
    mesh = plsc.VectorSubcoreMesh(core_axis_name="c", subcore_axis_name="s")

    @functools.partial(
        pl.kernel,
        out_type=jax.ShapeDtypeStruct((E,), jnp.float32),
        mesh=mesh,
        scratch_types=[
            pltpu.VMEM((N,), jnp.float32),
            pltpu.VMEM((CHA,), jnp.int32),
            pltpu.VMEM((CHA,), jnp.int32),
            pltpu.VMEM((CHA,), jnp.float32),
        ],
        compiler_params=_sc_compiler_params(),
    )
    def run(node_hbm, src_hbm, dst_hbm, mm_hbm, node_v, src_v, dst_v, mm_v):
        wid = lax.axis_index("s") * NC + lax.axis_index("c")
        base0 = wid * EPW
        pltpu.sync_copy(node_hbm, node_v)

        @pl.loop(0, EPW, step=CHA)
        def _(j):
            base = base0 + j
            pltpu.sync_copy(src_hbm.at[pl.ds(base, CHA)], src_v)
            pltpu.sync_copy(dst_hbm.at[pl.ds(base, CHA)], dst_v)

            @pl.loop(0, CHA, step=16)
            def _(g):
                sv = src_v[pl.ds(g, 16)]
                dv = dst_v[pl.ds(g, 16)]
                ms = plsc.load_gather(node_v, [sv])
                md = plsc.load_gather(node_v, [dv])
                mm_v[pl.ds(g, 16)] = ms * md

            pltpu.sync_copy(mm_v, mm_hbm.at[pl.ds(base, CHA)])

    return run(node_flat, src, dst)


CHD = 2000


def _deint(ea_flat, start, count):
    cnt = count // NW
    mesh = plsc.VectorSubcoreMesh(core_axis_name="c", subcore_axis_name="s")

    @functools.partial(
        pl.kernel,
        out_type=[jax.ShapeDtypeStruct((count,), jnp.float32),
                  jax.ShapeDtypeStruct((count,), jnp.float32)],
        mesh=mesh,
        scratch_types=[
            pltpu.VMEM((2 * CHD,), jnp.float32),
            pltpu.VMEM((CHD,), jnp.float32),
            pltpu.VMEM((CHD,), jnp.float32),
        ],
        compiler_params=_sc_compiler_params(),
    )
    def run(ea_hbm, xs_hbm, ys_hbm, buf_v, xs_v, ys_v):
        wid = lax.axis_index("s") * NC + lax.axis_index("c")
        base0 = wid * cnt
        iota2 = lax.iota(jnp.int32, 16) * 2

        @pl.loop(0, cnt, step=CHD)
        def _(j):
            base = base0 + j
            pltpu.sync_copy(ea_hbm.at[pl.ds((base + start) * 2, 2 * CHD)], buf_v)

            @pl.loop(0, CHD, step=16)
            def _(g):
                ix = iota2 + g * 2
                xs_v[pl.ds(g, 16)] = plsc.load_gather(buf_v, [ix])
                ys_v[pl.ds(g, 16)] = plsc.load_gather(buf_v, [ix + 1])

            pltpu.sync_copy(xs_v, xs_hbm.at[pl.ds(base, CHD)])
            pltpu.sync_copy(ys_v, ys_hbm.at[pl.ds(base, CHD)])

    return run(ea_flat)


GC = 2000
NB = E // (8 * GC)


def _mlp_body(x_ref, y_ref, w1b_ref, b1_ref, w2t_ref, b2_ref, w3t_ref, b3_ref,
              fx_ref, fy_ref):
    x8 = x_ref[0]
    y8 = y_ref[0]
    r2 = x8 * x8 + y8 * y8 + 1e-12
    logr = 0.5 * jnp.log(r2)
    theta = jnp.arctan2(y8, x8)
    s16 = jnp.concatenate([logr, theta], axis=0).astype(jnp.bfloat16)
    h1 = jnp.tanh(
        jnp.dot(w1b_ref[...], s16, preferred_element_type=jnp.float32)
        + b1_ref[...])
    h1b = h1.astype(jnp.bfloat16)
    lrs, ths = [], []
    for r in range(8):
        h2r = jnp.tanh(
            jnp.dot(w2t_ref[...], h1b[128 * r:128 * (r + 1)],
                    preferred_element_type=jnp.float32)
            + b2_ref[...])
        fr = (jnp.dot(w3t_ref[...], h2r, preferred_element_type=jnp.float32)
              + b3_ref[...])
        lrs.append(fr[0:1])
        ths.append(fr[1:2])
    lr8 = jnp.concatenate(lrs, axis=0)
    th8 = jnp.concatenate(ths, axis=0)
    rad = jnp.exp(lr8)
    fx_ref[0] = rad * jnp.cos(th8)
    fy_ref[0] = rad * jnp.sin(th8)


def _edge_mlp(xs3, ys3, w1big, b1big, w2tb, b2c, w3t, b3c):
    nb = xs3.shape[0]
    grid = (nb,)
    full = lambda shape: pl.BlockSpec(shape, lambda i: tuple(0 for _ in shape))
    return pl.pallas_call(
        _mlp_body,
        grid=grid,
        in_specs=[
            pl.BlockSpec((1, 8, GC), lambda i: (i, 0, 0)),
            pl.BlockSpec((1, 8, GC), lambda i: (i, 0, 0)),
            full((1024, 16)),
            full((1024, 1)),
            full((128, 128)),
            full((128, 1)),
            full((2, 128)),
            full((2, 1)),
        ],
        out_specs=[
            pl.BlockSpec((1, 8, GC), lambda i: (i, 0, 0)),
            pl.BlockSpec((1, 8, GC), lambda i: (i, 0, 0)),
        ],
        out_shape=[
            jax.ShapeDtypeStruct((nb, 8, GC), jnp.float32),
            jax.ShapeDtypeStruct((nb, 8, GC), jnp.float32),
        ],
    )(xs3, ys3, w1big, b1big, w2tb, b2c, w3t, b3c)


def _scatter_partials(dst, mm, fx, fy, zeros, start, count):
    cnt = count // NW
    mesh = plsc.VectorSubcoreMesh(core_axis_name="c", subcore_axis_name="s")

    @functools.partial(
        pl.kernel,
        out_type=jax.ShapeDtypeStruct((NW, NP2), jnp.float32),
        mesh=mesh,
        scratch_types=[
            pltpu.VMEM((NP2,), jnp.float32),
            pltpu.VMEM((CHB,), jnp.int32),
            pltpu.VMEM((CHB,), jnp.float32),
            pltpu.VMEM((CHB,), jnp.float32),
            pltpu.VMEM((CHB,), jnp.float32),
        ],
        compiler_params=_sc_compiler_params(),
    )
    def run(dst_hbm, mm_hbm, fx_hbm, fy_hbm, zero_hbm, out_hbm,
            acc_v, dst_v, mm_v, fx_v, fy_v):
        wid = lax.axis_index("s") * NC + lax.axis_index("c")
        base0 = wid * cnt
        pltpu.sync_copy(zero_hbm, acc_v)

        @pl.loop(0, cnt, step=CHB)
        def _(j):
            base = base0 + j
            pltpu.sync_copy(dst_hbm.at[pl.ds(base + start, CHB)], dst_v)
            pltpu.sync_copy(mm_hbm.at[pl.ds(base + start, CHB)], mm_v)
            pltpu.sync_copy(fx_hbm.at[pl.ds(base, CHB)], fx_v)
            pltpu.sync_copy(fy_hbm.at[pl.ds(base, CHB)], fy_v)

            @pl.loop(0, CHB, step=16)
            def _(g):
                d = dst_v[pl.ds(g, 16)]
                m = mm_v[pl.ds(g, 16)]
                vx = fx_v[pl.ds(g, 16)] * m
                vy = fy_v[pl.ds(g, 16)] * m
                plsc.addupdate_scatter(acc_v, [d], vx)
                plsc.addupdate_scatter(acc_v, [d + NPAD], vy)

        pltpu.sync_copy(acc_v, out_hbm.at[wid])

    return run(dst, mm, fx, fy, zeros)


SPLIT = 832000


def _reduce_body(p1_ref, p2_ref, o_ref):
    o_ref[...] = (jnp.sum(p1_ref[...], axis=0, keepdims=True)
                  + jnp.sum(p2_ref[...], axis=0, keepdims=True))


def _reduce_partials(p1, p2):
    grid = (NP2 // BTC,)
    return pl.pallas_call(
        _reduce_body,
        grid=grid,
        in_specs=[pl.BlockSpec((NW, BTC), lambda i: (0, i)),
                  pl.BlockSpec((NW, BTC), lambda i: (0, i))],
        out_specs=pl.BlockSpec((1, BTC), lambda i: (0, i)),
        out_shape=jax.ShapeDtypeStruct((1, NP2), jnp.float32),
    )(p1, p2)


def kernel(node_attr, edge_index, edge_attr, W1, b1, W2, b2, W3, b3):
    node_flat = node_attr.reshape(N)
    src = edge_index[0]
    dst = edge_index[1]
    ea_flat = edge_attr.reshape(2 * E)
    w1t = W1.T
    eye8 = jnp.eye(8, dtype=jnp.float32)
    w1big = jnp.concatenate(
        [jnp.kron(eye8, w1t[:, 0:1]), jnp.kron(eye8, w1t[:, 1:2])],
        axis=1).astype(jnp.bfloat16)
    b1big = jnp.tile(b1, 8).reshape(1024, 1)
    w2tb = W2.T.astype(jnp.bfloat16)
    b2c = b2.reshape(128, 1)
    w3t = W3.T
    b3c = b3.reshape(2, 1)
    zeros = jnp.zeros((NP2,), jnp.float32)

    e2 = E - SPLIT
    nb1, nb2 = SPLIT // (8 * GC), e2 // (8 * GC)
    xs1, ys1 = _deint(ea_flat, 0, SPLIT)
    xs2, ys2 = _deint(ea_flat, SPLIT, e2)
    mm = _mass_product(node_flat, src, dst)
    fx1, fy1 = _edge_mlp(xs1.reshape(nb1, 8, GC), ys1.reshape(nb1, 8, GC),
                         w1big, b1big, w2tb, b2c, w3t, b3c)
    fx2, fy2 = _edge_mlp(xs2.reshape(nb2, 8, GC), ys2.reshape(nb2, 8, GC),
                         w1big, b1big, w2tb, b2c, w3t, b3c)
    p1 = _scatter_partials(dst, mm, fx1.reshape(SPLIT), fy1.reshape(SPLIT),
                           zeros, 0, SPLIT)
    p2 = _scatter_partials(dst, mm, fx2.reshape(e2), fy2.reshape(e2),
                           zeros, SPLIT, e2)
    red = _reduce_partials(p1, p2)
    out = jnp.stack([red[0, :N], red[0, NPAD:NPAD + N]], axis=1)
    return out


# speedup vs baseline: 3.8922x; 3.8922x over previous
import dataclasses
import functools

import jax
import jax.numpy as jnp
from jax import lax
from jax.experimental import pallas as pl
from jax.experimental.pallas import tpu as pltpu
from jax.experimental.pallas import tpu_sc as plsc

N = 50000
E = 1600000
NC = 2
NS = 16
NW = NC * NS
EPW = E // NW
CHA = 2000
CHB = 2000
NPAD = 50176
NP2 = 2 * NPAD
BT = 6400
BTC = 6272


def _sc_compiler_params():
    cp = pltpu.CompilerParams()
    if "needs_layout_passes" in pltpu.CompilerParams.__dataclass_fields__:
        cp = dataclasses.replace(cp, needs_layout_passes=False)
    return cp


def _mass_product(node_flat, src, dst):
    mesh = plsc.VectorSubcoreMesh(core_axis_name="c", subcore_axis_name="s")

    @functools.partial(
        pl.kernel,
        out_type=jax.ShapeDtypeStruct((E,), jnp.float32),
        mesh=mesh,
        scratch_types=[
            pltpu.VMEM((N,), jnp.float32),
            pltpu.VMEM((CHA,), jnp.int32),
            pltpu.VMEM((CHA,), jnp.int32),
            pltpu.VMEM((CHA,), jnp.float32),
        ],
        compiler_params=_sc_compiler_params(),
    )
    def run(node_hbm, src_hbm, dst_hbm, mm_hbm, node_v, src_v, dst_v, mm_v):
        wid = lax.axis_index("s") * NC + lax.axis_index("c")
        base0 = wid * EPW
        pltpu.sync_copy(node_hbm, node_v)

        @pl.loop(0, EPW, step=CHA)
        def _(j):
            base = base0 + j
            pltpu.sync_copy(src_hbm.at[pl.ds(base, CHA)], src_v)
            pltpu.sync_copy(dst_hbm.at[pl.ds(base, CHA)], dst_v)

            @pl.loop(0, CHA, step=16)
            def _(g):
                sv = src_v[pl.ds(g, 16)]
                dv = dst_v[pl.ds(g, 16)]
                ms = plsc.load_gather(node_v, [sv])
                md = plsc.load_gather(node_v, [dv])
                mm_v[pl.ds(g, 16)] = ms * md

            pltpu.sync_copy(mm_v, mm_hbm.at[pl.ds(base, CHA)])

    return run(node_flat, src, dst)


CHD = 2000


def _deint(ea, start, count):
    cnt = count // NW
    mesh = plsc.VectorSubcoreMesh(core_axis_name="c", subcore_axis_name="s")

    @functools.partial(
        pl.kernel,
        out_type=[jax.ShapeDtypeStruct((count,), jnp.float32),
                  jax.ShapeDtypeStruct((count,), jnp.float32)],
        mesh=mesh,
        scratch_types=[
            pltpu.VMEM((CHD,), jnp.float32),
            pltpu.VMEM((CHD,), jnp.float32),
        ],
        compiler_params=_sc_compiler_params(),
    )
    def run(ea_hbm, xs_hbm, ys_hbm, xs_v, ys_v):
        wid = lax.axis_index("s") * NC + lax.axis_index("c")
        base0 = wid * cnt

        @pl.loop(0, cnt, step=CHD)
        def _(j):
            base = base0 + j
            pltpu.sync_copy(ea_hbm.at[pl.ds(base + start, CHD), 0], xs_v)
            pltpu.sync_copy(ea_hbm.at[pl.ds(base + start, CHD), 1], ys_v)
            pltpu.sync_copy(xs_v, xs_hbm.at[pl.ds(base, CHD)])
            pltpu.sync_copy(ys_v, ys_hbm.at[pl.ds(base, CHD)])

    return run(ea)


GC = 2000
NB = E // (8 * GC)


def _mlp_body(x_ref, y_ref, w1b_ref, b1_ref, w2t_ref, b2_ref, w3t_ref, b3_ref,
              fx_ref, fy_ref):
    x8 = x_ref[0]
    y8 = y_ref[0]
    r2 = x8 * x8 + y8 * y8 + 1e-12
    logr = 0.5 * jnp.log(r2)
    theta = jnp.arctan2(y8, x8)
    s16 = jnp.concatenate([logr, theta], axis=0).astype(jnp.bfloat16)
    h1 = jnp.tanh(
        jnp.dot(w1b_ref[...], s16, preferred_element_type=jnp.float32)
        + b1_ref[...])
    h1b = h1.astype(jnp.bfloat16)
    lrs, ths = [], []
    for r in range(8):
        h2r = jnp.tanh(
            jnp.dot(w2t_ref[...], h1b[128 * r:128 * (r + 1)],
                    preferred_element_type=jnp.float32)
            + b2_ref[...])
        fr = (jnp.dot(w3t_ref[...], h2r, preferred_element_type=jnp.float32)
              + b3_ref[...])
        lrs.append(fr[0:1])
        ths.append(fr[1:2])
    lr8 = jnp.concatenate(lrs, axis=0)
    th8 = jnp.concatenate(ths, axis=0)
    rad = jnp.exp(lr8)
    fx_ref[0] = rad * jnp.cos(th8)
    fy_ref[0] = rad * jnp.sin(th8)


def _edge_mlp(xs3, ys3, w1big, b1big, w2tb, b2c, w3t, b3c):
    nb = xs3.shape[0]
    grid = (nb,)
    full = lambda shape: pl.BlockSpec(shape, lambda i: tuple(0 for _ in shape))
    return pl.pallas_call(
        _mlp_body,
        grid=grid,
        in_specs=[
            pl.BlockSpec((1, 8, GC), lambda i: (i, 0, 0)),
            pl.BlockSpec((1, 8, GC), lambda i: (i, 0, 0)),
            full((1024, 16)),
            full((1024, 1)),
            full((128, 128)),
            full((128, 1)),
            full((2, 128)),
            full((2, 1)),
        ],
        out_specs=[
            pl.BlockSpec((1, 8, GC), lambda i: (i, 0, 0)),
            pl.BlockSpec((1, 8, GC), lambda i: (i, 0, 0)),
        ],
        out_shape=[
            jax.ShapeDtypeStruct((nb, 8, GC), jnp.float32),
            jax.ShapeDtypeStruct((nb, 8, GC), jnp.float32),
        ],
    )(xs3, ys3, w1big, b1big, w2tb, b2c, w3t, b3c)


def _scatter_partials(dst, mm, fx, fy, zeros, start, count):
    cnt = count // NW
    mesh = plsc.VectorSubcoreMesh(core_axis_name="c", subcore_axis_name="s")

    @functools.partial(
        pl.kernel,
        out_type=jax.ShapeDtypeStruct((NW, NP2), jnp.float32),
        mesh=mesh,
        scratch_types=[
            pltpu.VMEM((NP2,), jnp.float32),
            pltpu.VMEM((CHB,), jnp.int32),
            pltpu.VMEM((CHB,), jnp.float32),
            pltpu.VMEM((CHB,), jnp.float32),
            pltpu.VMEM((CHB,), jnp.float32),
        ],
        compiler_params=_sc_compiler_params(),
    )
    def run(dst_hbm, mm_hbm, fx_hbm, fy_hbm, zero_hbm, out_hbm,
            acc_v, dst_v, mm_v, fx_v, fy_v):
        wid = lax.axis_index("s") * NC + lax.axis_index("c")
        base0 = wid * cnt
        pltpu.sync_copy(zero_hbm, acc_v)

        @pl.loop(0, cnt, step=CHB)
        def _(j):
            base = base0 + j
            pltpu.sync_copy(dst_hbm.at[pl.ds(base + start, CHB)], dst_v)
            pltpu.sync_copy(mm_hbm.at[pl.ds(base + start, CHB)], mm_v)
            pltpu.sync_copy(fx_hbm.at[pl.ds(base, CHB)], fx_v)
            pltpu.sync_copy(fy_hbm.at[pl.ds(base, CHB)], fy_v)

            @pl.loop(0, CHB, step=16)
            def _(g):
                d = dst_v[pl.ds(g, 16)]
                m = mm_v[pl.ds(g, 16)]
                vx = fx_v[pl.ds(g, 16)] * m
                vy = fy_v[pl.ds(g, 16)] * m
                plsc.addupdate_scatter(acc_v, [d], vx)
                plsc.addupdate_scatter(acc_v, [d + NPAD], vy)

        pltpu.sync_copy(acc_v, out_hbm.at[wid])

    return run(dst, mm, fx, fy, zeros)


SPLIT = 832000


def _reduce_body(p1_ref, p2_ref, o_ref):
    o_ref[...] = (jnp.sum(p1_ref[...], axis=0, keepdims=True)
                  + jnp.sum(p2_ref[...], axis=0, keepdims=True))


def _reduce_partials(p1, p2):
    grid = (NP2 // BTC,)
    return pl.pallas_call(
        _reduce_body,
        grid=grid,
        in_specs=[pl.BlockSpec((NW, BTC), lambda i: (0, i)),
                  pl.BlockSpec((NW, BTC), lambda i: (0, i))],
        out_specs=pl.BlockSpec((1, BTC), lambda i: (0, i)),
        out_shape=jax.ShapeDtypeStruct((1, NP2), jnp.float32),
    )(p1, p2)


def kernel(node_attr, edge_index, edge_attr, W1, b1, W2, b2, W3, b3):
    node_flat = node_attr.reshape(N)
    w1t = W1.T
    eye8 = jnp.eye(8, dtype=jnp.float32)
    w1big = jnp.concatenate(
        [jnp.kron(eye8, w1t[:, 0:1]), jnp.kron(eye8, w1t[:, 1:2])],
        axis=1).astype(jnp.bfloat16)
    b1big = jnp.tile(b1, 8).reshape(1024, 1)
    w2tb = W2.T.astype(jnp.bfloat16)
    b2c = b2.reshape(128, 1)
    w3t = W3.T
    b3c = b3.reshape(2, 1)
    zeros = jnp.zeros((NP2,), jnp.float32)

    e2 = E - SPLIT
    nb1, nb2 = SPLIT // (8 * GC), e2 // (8 * GC)
    xs1 = edge_attr[:SPLIT, 0].reshape(nb1, 8, GC)
    ys1 = edge_attr[:SPLIT, 1].reshape(nb1, 8, GC)
    xs2 = edge_attr[SPLIT:, 0].reshape(nb2, 8, GC)
    ys2 = edge_attr[SPLIT:, 1].reshape(nb2, 8, GC)
    src = edge_index[0]
    dst = edge_index[1]
    mm = _mass_product(node_flat, src, dst)
    fx1, fy1 = _edge_mlp(xs1, ys1, w1big, b1big, w2tb, b2c, w3t, b3c)
    fx2, fy2 = _edge_mlp(xs2, ys2, w1big, b1big, w2tb, b2c, w3t, b3c)
    p1 = _scatter_partials(dst, mm, fx1.reshape(SPLIT),
                           fy1.reshape(SPLIT), zeros, 0, SPLIT)
    p2 = _scatter_partials(dst, mm, fx2.reshape(e2), fy2.reshape(e2),
                           zeros, SPLIT, e2)
    red = _reduce_partials(p1, p2)
    out = jnp.stack([red[0, :N], red[0, NPAD:NPAD + N]], axis=1)
    return out


# speedup vs baseline: 3.9074x; 1.0039x over previous
import dataclasses
import functools

import jax
import jax.numpy as jnp
from jax import lax
from jax.experimental import pallas as pl
from jax.experimental.pallas import tpu as pltpu
from jax.experimental.pallas import tpu_sc as plsc

N = 50000
E = 1600000
NC = 2
NS = 16
NW = NC * NS
EPW = E // NW
CHA = 2000
CHB = 2000
NPAD = 50176
NP2 = 2 * NPAD
BT = 6400
BTC = 6272


def _sc_compiler_params():
    cp = pltpu.CompilerParams()
    if "needs_layout_passes" in pltpu.CompilerParams.__dataclass_fields__:
        cp = dataclasses.replace(cp, needs_layout_passes=False)
    return cp


def _mass_product(node_flat, src, dst):
    mesh = plsc.VectorSubcoreMesh(core_axis_name="c", subcore_axis_name="s")

    @functools.partial(
        pl.kernel,
        out_type=jax.ShapeDtypeStruct((E,), jnp.float32),
        mesh=mesh,
        scratch_types=[
            pltpu.VMEM((N,), jnp.float32),
            pltpu.VMEM((CHA,), jnp.int32),
            pltpu.VMEM((CHA,), jnp.int32),
            pltpu.VMEM((CHA,), jnp.float32),
        ],
        compiler_params=_sc_compiler_params(),
    )
    def run(node_hbm, src_hbm, dst_hbm, mm_hbm, node_v, src_v, dst_v, mm_v):
        wid = lax.axis_index("s") * NC + lax.axis_index("c")
        base0 = wid * EPW
        pltpu.sync_copy(node_hbm, node_v)

        @pl.loop(0, EPW, step=CHA)
        def _(j):
            base = base0 + j
            pltpu.sync_copy(src_hbm.at[pl.ds(base, CHA)], src_v)
            pltpu.sync_copy(dst_hbm.at[pl.ds(base, CHA)], dst_v)

            @pl.loop(0, CHA, step=16)
            def _(g):
                sv = src_v[pl.ds(g, 16)]
                dv = dst_v[pl.ds(g, 16)]
                ms = plsc.load_gather(node_v, [sv])
                md = plsc.load_gather(node_v, [dv])
                mm_v[pl.ds(g, 16)] = ms * md

            pltpu.sync_copy(mm_v, mm_hbm.at[pl.ds(base, CHA)])

    return run(node_flat, src, dst)


CHD = 2000


def _deint(ea, start, count):
    cnt = count // NW
    mesh = plsc.VectorSubcoreMesh(core_axis_name="c", subcore_axis_name="s")

    @functools.partial(
        pl.kernel,
        out_type=[jax.ShapeDtypeStruct((count,), jnp.float32),
                  jax.ShapeDtypeStruct((count,), jnp.float32)],
        mesh=mesh,
        scratch_types=[
            pltpu.VMEM((CHD,), jnp.float32),
            pltpu.VMEM((CHD,), jnp.float32),
        ],
        compiler_params=_sc_compiler_params(),
    )
    def run(ea_hbm, xs_hbm, ys_hbm, xs_v, ys_v):
        wid = lax.axis_index("s") * NC + lax.axis_index("c")
        base0 = wid * cnt

        @pl.loop(0, cnt, step=CHD)
        def _(j):
            base = base0 + j
            pltpu.sync_copy(ea_hbm.at[pl.ds(base + start, CHD), 0], xs_v)
            pltpu.sync_copy(ea_hbm.at[pl.ds(base + start, CHD), 1], ys_v)
            pltpu.sync_copy(xs_v, xs_hbm.at[pl.ds(base, CHD)])
            pltpu.sync_copy(ys_v, ys_hbm.at[pl.ds(base, CHD)])

    return run(ea)


GC = 2000
NB = E // (8 * GC)


def _mlp_body(x_ref, y_ref, w1b_ref, b1_ref, w2t_ref, b2_ref, w3t_ref, b3_ref,
              fx_ref, fy_ref):
    x8 = x_ref[0]
    y8 = y_ref[0]
    r2 = x8 * x8 + y8 * y8 + 1e-12
    logr = 0.5 * jnp.log(r2)
    theta = jnp.arctan2(y8, x8)
    s16 = jnp.concatenate([logr, theta], axis=0).astype(jnp.bfloat16)
    h1 = jnp.tanh(
        jnp.dot(w1b_ref[...], s16, preferred_element_type=jnp.float32)
        + b1_ref[...])
    h1b = h1.astype(jnp.bfloat16)
    lrs, ths = [], []
    for r in range(8):
        h2r = jnp.tanh(
            jnp.dot(w2t_ref[...], h1b[128 * r:128 * (r + 1)],
                    preferred_element_type=jnp.float32)
            + b2_ref[...])
        fr = (jnp.dot(w3t_ref[...], h2r, preferred_element_type=jnp.float32)
              + b3_ref[...])
        lrs.append(fr[0:1])
        ths.append(fr[1:2])
    lr8 = jnp.concatenate(lrs, axis=0)
    th8 = jnp.concatenate(ths, axis=0)
    rad = jnp.exp(lr8)
    fx_ref[0] = rad * jnp.cos(th8)
    fy_ref[0] = rad * jnp.sin(th8)


def _edge_mlp(xs3, ys3, w1big, b1big, w2tb, b2c, w3t, b3c):
    nb = xs3.shape[0]
    grid = (nb,)
    full = lambda shape: pl.BlockSpec(shape, lambda i: tuple(0 for _ in shape))
    return pl.pallas_call(
        _mlp_body,
        grid=grid,
        in_specs=[
            pl.BlockSpec((1, 8, GC), lambda i: (i, 0, 0)),
            pl.BlockSpec((1, 8, GC), lambda i: (i, 0, 0)),
            full((1024, 16)),
            full((1024, 1)),
            full((128, 128)),
            full((128, 1)),
            full((2, 128)),
            full((2, 1)),
        ],
        out_specs=[
            pl.BlockSpec((1, 8, GC), lambda i: (i, 0, 0)),
            pl.BlockSpec((1, 8, GC), lambda i: (i, 0, 0)),
        ],
        out_shape=[
            jax.ShapeDtypeStruct((nb, 8, GC), jnp.float32),
            jax.ShapeDtypeStruct((nb, 8, GC), jnp.float32),
        ],
    )(xs3, ys3, w1big, b1big, w2tb, b2c, w3t, b3c)


def _scatter_partials(dst, mm, fx, fy, zeros, start, count):
    cnt = count // NW
    mesh = plsc.VectorSubcoreMesh(core_axis_name="c", subcore_axis_name="s")

    @functools.partial(
        pl.kernel,
        out_type=jax.ShapeDtypeStruct((NW, NP2), jnp.float32),
        mesh=mesh,
        scratch_types=[
            pltpu.VMEM((NP2,), jnp.float32),
            pltpu.VMEM((CHB,), jnp.int32),
            pltpu.VMEM((CHB,), jnp.float32),
            pltpu.VMEM((CHB,), jnp.float32),
            pltpu.VMEM((CHB,), jnp.float32),
        ],
        compiler_params=_sc_compiler_params(),
    )
    def run(dst_hbm, mm_hbm, fx_hbm, fy_hbm, zero_hbm, out_hbm,
            acc_v, dst_v, mm_v, fx_v, fy_v):
        wid = lax.axis_index("s") * NC + lax.axis_index("c")
        base0 = wid * cnt
        pltpu.sync_copy(zero_hbm, acc_v)

        @pl.loop(0, cnt, step=CHB)
        def _(j):
            base = base0 + j
            pltpu.sync_copy(dst_hbm.at[pl.ds(base + start, CHB)], dst_v)
            pltpu.sync_copy(mm_hbm.at[pl.ds(base + start, CHB)], mm_v)
            pltpu.sync_copy(fx_hbm.at[pl.ds(base, CHB)], fx_v)
            pltpu.sync_copy(fy_hbm.at[pl.ds(base, CHB)], fy_v)

            @pl.loop(0, CHB, step=16)
            def _(g):
                d = dst_v[pl.ds(g, 16)]
                m = mm_v[pl.ds(g, 16)]
                vx = fx_v[pl.ds(g, 16)] * m
                vy = fy_v[pl.ds(g, 16)] * m
                plsc.addupdate_scatter(acc_v, [d], vx)
                plsc.addupdate_scatter(acc_v, [d + NPAD], vy)

        pltpu.sync_copy(acc_v, out_hbm.at[wid])

    return run(dst, mm, fx, fy, zeros)


CHUNK_SIZES = (448000, 384000, 384000, 384000)


def _reduce_partials(ps):
    n = len(ps)

    def body(*refs):
        o_ref = refs[-1]
        acc = jnp.sum(refs[0][...], axis=0, keepdims=True)
        for r in refs[1:-1]:
            acc = acc + jnp.sum(r[...], axis=0, keepdims=True)
        o_ref[...] = acc

    grid = (NP2 // BTC,)
    return pl.pallas_call(
        body,
        grid=grid,
        in_specs=[pl.BlockSpec((NW, BTC), lambda i: (0, i))] * n,
        out_specs=pl.BlockSpec((1, BTC), lambda i: (0, i)),
        out_shape=jax.ShapeDtypeStruct((1, NP2), jnp.float32),
    )(*ps)


def kernel(node_attr, edge_index, edge_attr, W1, b1, W2, b2, W3, b3):
    node_flat = node_attr.reshape(N)
    w1t = W1.T
    eye8 = jnp.eye(8, dtype=jnp.float32)
    w1big = jnp.concatenate(
        [jnp.kron(eye8, w1t[:, 0:1]), jnp.kron(eye8, w1t[:, 1:2])],
        axis=1).astype(jnp.bfloat16)
    b1big = jnp.tile(b1, 8).reshape(1024, 1)
    w2tb = W2.T.astype(jnp.bfloat16)
    b2c = b2.reshape(128, 1)
    w3t = W3.T
    b3c = b3.reshape(2, 1)
    zeros = jnp.zeros((NP2,), jnp.float32)

    src = edge_index[0]
    dst = edge_index[1]
    mm = _mass_product(node_flat, src, dst)
    partials = []
    start = 0
    for cs in CHUNK_SIZES:
        nb = cs // (8 * GC)
        xs = edge_attr[start:start + cs, 0].reshape(nb, 8, GC)
        ys = edge_attr[start:start + cs, 1].reshape(nb, 8, GC)
        fx, fy = _edge_mlp(xs, ys, w1big, b1big, w2tb, b2c, w3t, b3c)
        partials.append(_scatter_partials(dst, mm, fx.reshape(cs),
                                          fy.reshape(cs), zeros, start, cs))
        start += cs
    red = _reduce_partials(partials)
    out = jnp.stack([red[0, :N], red[0, NPAD:NPAD + N]], axis=1)
    return out
